# Initial kernel scaffold; baseline (speedup 1.0000x reference)
import jax
import jax.numpy as jnp
from jax.experimental import pallas as pl


def kernel(xyz, points, w0, b0, gamma0, beta0, w1, b1, gamma1, beta1, w2, b2, gamma2, beta2):
    raise NotImplementedError("write your pallas kernel here")


# speedup vs baseline: 3.9837x; 3.9837x over previous
import functools

import jax
import jax.numpy as jnp
from jax import lax
from jax.experimental import pallas as pl
from jax.experimental.pallas import tpu as pltpu
from jax.experimental.pallas import tpu_sc as plsc

_B, _N, _P, _S = 8, 8192, 1024, 32
_R2 = 0.4 * 0.4
_EPS = 1e-5
_NSUB, _NLANE = 8, 1024
_TOT = _B * _P * _S
_D = 32
_ROWS = 512
_GRID1 = _TOT // _ROWS
_NW = 32
_CHUNK = 128


def _fps_body(xyz_ref, ctr_ref):
    x = xyz_ref[0, 0]
    y = xyz_ref[0, 1]
    z = xyz_ref[0, 2]
    iota_n = (lax.broadcasted_iota(jnp.int32, (_NSUB, _NLANE), 0) * _NLANE
              + lax.broadcasted_iota(jnp.int32, (_NSUB, _NLANE), 1))
    iota_rec = (lax.broadcasted_iota(jnp.int32, (8, 128), 0) * 128
                + lax.broadcasted_iota(jnp.int32, (8, 128), 1))

    def body(i, st):
        dists, far, ax, ay, az = st
        oh = iota_n == far
        cx = jnp.sum(jnp.where(oh, x, 0.0))
        cy = jnp.sum(jnp.where(oh, y, 0.0))
        cz = jnp.sum(jnp.where(oh, z, 0.0))
        rec = iota_rec == i
        ax = jnp.where(rec, cx, ax)
        ay = jnp.where(rec, cy, ay)
        az = jnp.where(rec, cz, az)
        dx = x - cx
        dy = y - cy
        dz = z - cz
        d = dx * dx + dy * dy + dz * dz
        dists = jnp.minimum(dists, d)
        m = jnp.max(dists)
        far = jnp.min(jnp.where(dists == m, iota_n, _N)).astype(jnp.int32)
        return dists, far, ax, ay, az

    init = (jnp.full((_NSUB, _NLANE), 1e10, jnp.float32), jnp.int32(0),
            jnp.zeros((8, 128), jnp.float32),
            jnp.zeros((8, 128), jnp.float32),
            jnp.zeros((8, 128), jnp.float32))
    _, _, ax, ay, az = lax.fori_loop(0, _P, body, init)
    ctr_ref[0, 0] = ax
    ctr_ref[0, 1] = ay
    ctr_ref[0, 2] = az


def _run_fps(xyz):
    xyz_r = xyz.reshape(_B, 3, _NSUB, _NLANE)
    ctr = pl.pallas_call(
        _fps_body,
        grid=(_B,),
        in_specs=[pl.BlockSpec((1, 3, _NSUB, _NLANE), lambda b: (b, 0, 0, 0))],
        out_specs=pl.BlockSpec((1, 3, 8, 128), lambda b: (b, 0, 0, 0)),
        out_shape=jax.ShapeDtypeStruct((_B, 3, 8, 128), jnp.float32),
    )(xyz_r)
    return ctr.reshape(_B, 3, _P)


def _bq_body(xyz_ref, ctr_ref, idx_ref):
    b = pl.program_id(0)
    x = xyz_ref[0, 0:1, :]
    y = xyz_ref[0, 1:2, :]
    z = xyz_ref[0, 2:3, :]
    cx = ctr_ref[0, :, 0:1]
    cy = ctr_ref[0, :, 1:2]
    cz = ctr_ref[0, :, 2:3]
    xx = x * x + y * y + z * z
    cc = cx * cx + cy * cy + cz * cz
    xb = x.astype(jnp.bfloat16).astype(jnp.float32)
    yb = y.astype(jnp.bfloat16).astype(jnp.float32)
    zb = z.astype(jnp.bfloat16).astype(jnp.float32)
    cxb = cx.astype(jnp.bfloat16).astype(jnp.float32)
    cyb = cy.astype(jnp.bfloat16).astype(jnp.float32)
    czb = cz.astype(jnp.bfloat16).astype(jnp.float32)
    cross = cxb * xb + cyb * yb + czb * zb
    d2 = (cc + xx) - 2.0 * cross
    iota = lax.broadcasted_iota(jnp.int32, (8, _N), 1)
    cand = jnp.where(d2 <= _R2, iota, _N)
    cols = []
    for _ in range(_S):
        m = jnp.min(cand, axis=1, keepdims=True)
        cols.append(m)
        cand = jnp.where(cand == m, _N, cand)
    idxt = jnp.concatenate(cols, axis=1)
    idxt = jnp.where(idxt == _N, idxt[:, 0:1], idxt)
    idxt = jnp.minimum(idxt, _N - 1)
    idx_ref[0] = idxt + b * _N


def _run_ball_query(xyz, ctr_t):
    return pl.pallas_call(
        _bq_body,
        grid=(_B, _P // 8),
        in_specs=[
            pl.BlockSpec((1, 3, _N), lambda b, t: (b, 0, 0)),
            pl.BlockSpec((1, 8, 3), lambda b, t: (b, t, 0)),
        ],
        out_specs=pl.BlockSpec((1, 8, _S), lambda b, t: (b, t, 0)),
        out_shape=jax.ShapeDtypeStruct((_B, _P, _S), jnp.int32),
    )(xyz, ctr_t)


def _sc_gather_body(tab_ref, idx_ref, out_ref, idx_v, rows_v, sem):
    wid = lax.axis_index("s") * 2 + lax.axis_index("c")
    per_w = _TOT // _NW
    base = wid * per_w

    def chunk(c, carry):
        off = base + c * _CHUNK
        pltpu.sync_copy(idx_ref.at[pl.ds(off, _CHUNK)], idx_v)
        pltpu.async_copy(tab_ref.at[idx_v], rows_v, sem).wait()
        pltpu.sync_copy(rows_v, out_ref.at[pl.ds(off, _CHUNK)])
        return carry

    lax.fori_loop(0, per_w // _CHUNK, chunk, 0)


def _sc_gather(table, flat_idx):
    mesh = plsc.VectorSubcoreMesh(core_axis_name="c", subcore_axis_name="s")
    f = functools.partial(
        pl.kernel,
        mesh=mesh,
        out_type=jax.ShapeDtypeStruct((_TOT, 128), jnp.float32),
        scratch_types=[
            pltpu.VMEM((_CHUNK,), jnp.int32),
            pltpu.VMEM((_CHUNK, 128), jnp.float32),
            pltpu.SemaphoreType.DMA,
        ],
    )(_sc_gather_body)
    return f(table, flat_idx)


def _mlp1_body(g_ref, c_ref, w_ref, b_ref, y_ref, st_ref, acc):
    i = pl.program_id(0)

    @pl.when(i == 0)
    def _():
        acc[...] = jnp.zeros_like(acc)

    g = g_ref[...][:, 0:_D]
    c = c_ref[...]
    ce = jnp.reshape(jnp.broadcast_to(c[:, None, :], (_ROWS // _S, _S, 3)),
                     (_ROWS, 3))
    xcat = jnp.concatenate([g[:, 0:3] - ce, g[:, 3:]], axis=1)
    yv = jnp.dot(xcat.astype(jnp.bfloat16), w_ref[...].astype(jnp.bfloat16),
                 preferred_element_type=jnp.float32) + b_ref[...]
    y_ref[...] = yv
    acc[0:1, :] = acc[0:1, :] + jnp.sum(yv, axis=0, keepdims=True)
    acc[1:2, :] = acc[1:2, :] + jnp.sum(yv * yv, axis=0, keepdims=True)

    @pl.when(i == _GRID1 - 1)
    def _():
        st_ref[...] = acc[...]


def _run_mlp1(g, ctr_rows, w0p, b0):
    return pl.pallas_call(
        _mlp1_body,
        grid=(_GRID1,),
        in_specs=[
            pl.BlockSpec((_ROWS, 128), lambda i: (i, 0)),
            pl.BlockSpec((_ROWS // _S, 3), lambda i: (i, 0)),
            pl.BlockSpec((_D, _D), lambda i: (0, 0)),
            pl.BlockSpec((1, _D), lambda i: (0, 0)),
        ],
        out_specs=[
            pl.BlockSpec((_ROWS, _D), lambda i: (i, 0)),
            pl.BlockSpec((8, _D), lambda i: (0, 0)),
        ],
        out_shape=[
            jax.ShapeDtypeStruct((_TOT, _D), jnp.float32),
            jax.ShapeDtypeStruct((8, _D), jnp.float32),
        ],
        scratch_shapes=[pltpu.VMEM((8, _D), jnp.float32)],
    )(g, ctr_rows, w0p, b0)


def _mlp2_body(y_ref, a_ref, d_ref, w_ref, b_ref, o_ref, st_ref, acc):
    i = pl.program_id(0)

    @pl.when(i == 0)
    def _():
        acc[...] = jnp.zeros_like(acc)

    z = jnp.maximum(y_ref[...] * a_ref[...] + d_ref[...], 0.0)
    yv = jnp.dot(z.astype(jnp.bfloat16), w_ref[...].astype(jnp.bfloat16),
                 preferred_element_type=jnp.float32) + b_ref[...]
    o_ref[...] = yv
    acc[0:1, :] = acc[0:1, :] + jnp.sum(yv, axis=0, keepdims=True)
    acc[1:2, :] = acc[1:2, :] + jnp.sum(yv * yv, axis=0, keepdims=True)

    @pl.when(i == _GRID1 - 1)
    def _():
        st_ref[...] = acc[...]


def _run_mlp2(y0, a0, d0, w1p, b1):
    return pl.pallas_call(
        _mlp2_body,
        grid=(_GRID1,),
        in_specs=[
            pl.BlockSpec((_ROWS, _D), lambda i: (i, 0)),
            pl.BlockSpec((1, _D), lambda i: (0, 0)),
            pl.BlockSpec((1, _D), lambda i: (0, 0)),
            pl.BlockSpec((_D, _D), lambda i: (0, 0)),
            pl.BlockSpec((1, _D), lambda i: (0, 0)),
        ],
        out_specs=[
            pl.BlockSpec((_ROWS, _D), lambda i: (i, 0)),
            pl.BlockSpec((8, _D), lambda i: (0, 0)),
        ],
        out_shape=[
            jax.ShapeDtypeStruct((_TOT, _D), jnp.float32),
            jax.ShapeDtypeStruct((8, _D), jnp.float32),
        ],
        scratch_shapes=[pltpu.VMEM((8, _D), jnp.float32)],
    )(y0, a0, d0, w1p, b1)


def _mlp3_body(y_ref, a_ref, d_ref, w_ref, b_ref, mx_ref, mn_ref, st_ref, acc):
    i = pl.program_id(0)

    @pl.when(i == 0)
    def _():
        acc[...] = jnp.zeros_like(acc)

    z = jnp.maximum(y_ref[...] * a_ref[...] + d_ref[...], 0.0)
    yv = jnp.dot(z.astype(jnp.bfloat16), w_ref[...].astype(jnp.bfloat16),
                 preferred_element_type=jnp.float32) + b_ref[...]
    acc[0:1, :] = acc[0:1, :] + jnp.sum(yv, axis=0, keepdims=True)
    acc[1:2, :] = acc[1:2, :] + jnp.sum(yv * yv, axis=0, keepdims=True)
    yr = yv.reshape(_ROWS // _S, _S, 64)
    mx_ref[...] = jnp.max(yr, axis=1)
    mn_ref[...] = jnp.min(yr, axis=1)

    @pl.when(i == _GRID1 - 1)
    def _():
        st_ref[...] = acc[...]


def _run_mlp3(y1, a1, d1, w2p, b2):
    return pl.pallas_call(
        _mlp3_body,
        grid=(_GRID1,),
        in_specs=[
            pl.BlockSpec((_ROWS, _D), lambda i: (i, 0)),
            pl.BlockSpec((1, _D), lambda i: (0, 0)),
            pl.BlockSpec((1, _D), lambda i: (0, 0)),
            pl.BlockSpec((_D, 64), lambda i: (0, 0)),
            pl.BlockSpec((1, 64), lambda i: (0, 0)),
        ],
        out_specs=[
            pl.BlockSpec((_ROWS // _S, 64), lambda i: (i, 0)),
            pl.BlockSpec((_ROWS // _S, 64), lambda i: (i, 0)),
            pl.BlockSpec((8, 64), lambda i: (0, 0)),
        ],
        out_shape=[
            jax.ShapeDtypeStruct((_B * _P, 64), jnp.float32),
            jax.ShapeDtypeStruct((_B * _P, 64), jnp.float32),
            jax.ShapeDtypeStruct((8, 64), jnp.float32),
        ],
        scratch_shapes=[pltpu.VMEM((8, 64), jnp.float32)],
    )(y1, a1, d1, w2p, b2)


def _fin_body(mx_ref, mn_ref, g_ref, a_ref, d_ref, o_ref):
    g = g_ref[...]
    sel = jnp.where(g >= 0.0, mx_ref[...], mn_ref[...])
    o_ref[...] = jnp.maximum(sel * a_ref[...] + d_ref[...], 0.0)


def _run_final(mx, mn, g2, a2, d2):
    return pl.pallas_call(
        _fin_body,
        grid=(16,),
        in_specs=[
            pl.BlockSpec((_ROWS, 64), lambda i: (i, 0)),
            pl.BlockSpec((_ROWS, 64), lambda i: (i, 0)),
            pl.BlockSpec((1, 64), lambda i: (0, 0)),
            pl.BlockSpec((1, 64), lambda i: (0, 0)),
            pl.BlockSpec((1, 64), lambda i: (0, 0)),
        ],
        out_specs=pl.BlockSpec((_ROWS, 64), lambda i: (i, 0)),
        out_shape=jax.ShapeDtypeStruct((_B * _P, 64), jnp.float32),
    )(mx, mn, g2, a2, d2)


def _bn_affine(s, ss, gamma, beta):
    cnt = float(_TOT)
    mean = s / cnt
    var = jnp.maximum(ss / cnt - mean * mean, 0.0)
    a = gamma / jnp.sqrt(var + _EPS)
    d = beta - mean * a
    return a, d


def kernel(xyz, points, w0, b0, gamma0, beta0, w1, b1, gamma1, beta1,
           w2, b2, gamma2, beta2):
    new_xyz = _run_fps(xyz)
    ctr_t = jnp.transpose(new_xyz, (0, 2, 1))

    idx = _run_ball_query(xyz, ctr_t)

    table = jnp.concatenate(
        [jnp.transpose(xyz, (0, 2, 1)),
         jnp.transpose(points, (0, 2, 1)),
         jnp.zeros((_B, _N, 128 - 19), jnp.float32)], axis=-1
    ).reshape(_B * _N, 128)
    g = _sc_gather(table, idx.reshape(_TOT))

    w0p = jnp.zeros((_D, _D), jnp.float32).at[:19, :].set(w0.T)
    w1p = w1.T
    w2p = w2.T
    ctr_rows = ctr_t.reshape(_B * _P, 3)
    y0, st0 = _run_mlp1(g, ctr_rows, w0p, b0.reshape(1, _D))
    a0, d0 = _bn_affine(st0[0], st0[1], gamma0, beta0)
    y1, st1 = _run_mlp2(y0, a0.reshape(1, _D), d0.reshape(1, _D),
                        w1p, b1.reshape(1, _D))
    a1, d1 = _bn_affine(st1[0], st1[1], gamma1, beta1)
    mx, mn, st2 = _run_mlp3(y1, a1.reshape(1, _D), d1.reshape(1, _D),
                            w2p, b2.reshape(1, 64))
    a2, d2 = _bn_affine(st2[0, :], st2[1, :], gamma2, beta2)
    pts = _run_final(mx, mn, gamma2.reshape(1, 64), a2.reshape(1, 64),
                     d2.reshape(1, 64))
    new_points = jnp.transpose(pts.reshape(_B, _P, 64), (0, 2, 1))
    return new_xyz, new_points


# speedup vs baseline: 5.4155x; 1.3594x over previous
import functools

import jax
import jax.numpy as jnp
from jax import lax
from jax.experimental import pallas as pl
from jax.experimental.pallas import tpu as pltpu
from jax.experimental.pallas import tpu_sc as plsc

_B, _N, _P, _S = 8, 8192, 1024, 32
_R2 = 0.4 * 0.4
_EPS = 1e-5
_NSUB, _NLANE = 8, 1024
_TOT = _B * _P * _S
_D = 32
_ROWS = 512
_GRID1 = _TOT // _ROWS
_NW = 32
_CHUNK = 128


def _fps_body(xyz_ref, ctr_ref):
    x = xyz_ref[0]
    y = xyz_ref[1]
    z = xyz_ref[2]
    iota_n = lax.broadcasted_iota(jnp.int32, (_B, _N), 1)
    iota_p = lax.broadcasted_iota(jnp.int32, (_B, _P), 1)

    def body(i, st):
        dists, far, ax, ay, az = st
        oh = iota_n == far
        cx = jnp.sum(jnp.where(oh, x, 0.0), axis=1, keepdims=True)
        cy = jnp.sum(jnp.where(oh, y, 0.0), axis=1, keepdims=True)
        cz = jnp.sum(jnp.where(oh, z, 0.0), axis=1, keepdims=True)
        rec = iota_p == i
        ax = jnp.where(rec, cx, ax)
        ay = jnp.where(rec, cy, ay)
        az = jnp.where(rec, cz, az)
        dx = x - cx
        dy = y - cy
        dz = z - cz
        d = dx * dx + dy * dy + dz * dz
        dists = jnp.minimum(dists, d)
        m = jnp.max(dists, axis=1, keepdims=True)
        far = jnp.min(jnp.where(dists == m, iota_n, _N), axis=1,
                      keepdims=True).astype(jnp.int32)
        return dists, far, ax, ay, az

    init = (jnp.full((_B, _N), 1e10, jnp.float32),
            jnp.zeros((_B, 1), jnp.int32),
            jnp.zeros((_B, _P), jnp.float32),
            jnp.zeros((_B, _P), jnp.float32),
            jnp.zeros((_B, _P), jnp.float32))
    _, _, ax, ay, az = lax.fori_loop(0, _P, body, init)
    ctr_ref[0] = ax
    ctr_ref[1] = ay
    ctr_ref[2] = az


def _run_fps(xyz):
    xyz_sb = jnp.transpose(xyz, (1, 0, 2))
    ctr = pl.pallas_call(
        _fps_body,
        grid=(1,),
        in_specs=[pl.BlockSpec((3, _B, _N), lambda i: (0, 0, 0))],
        out_specs=pl.BlockSpec((3, _B, _P), lambda i: (0, 0, 0)),
        out_shape=jax.ShapeDtypeStruct((3, _B, _P), jnp.float32),
    )(xyz_sb)
    return jnp.transpose(ctr, (1, 0, 2))


def _bq_body(xyz_ref, ctr_ref, idx_ref):
    b = pl.program_id(0)
    x = xyz_ref[0, 0:1, :]
    y = xyz_ref[0, 1:2, :]
    z = xyz_ref[0, 2:3, :]
    cx = ctr_ref[0, :, 0:1]
    cy = ctr_ref[0, :, 1:2]
    cz = ctr_ref[0, :, 2:3]
    xx = x * x + y * y + z * z
    cc = cx * cx + cy * cy + cz * cz
    xb = x.astype(jnp.bfloat16).astype(jnp.float32)
    yb = y.astype(jnp.bfloat16).astype(jnp.float32)
    zb = z.astype(jnp.bfloat16).astype(jnp.float32)
    cxb = cx.astype(jnp.bfloat16).astype(jnp.float32)
    cyb = cy.astype(jnp.bfloat16).astype(jnp.float32)
    czb = cz.astype(jnp.bfloat16).astype(jnp.float32)
    cross = cxb * xb + cyb * yb + czb * zb
    d2 = (cc + xx) - 2.0 * cross
    iota = lax.broadcasted_iota(jnp.int32, (8, _N), 1)
    cand = jnp.where(d2 <= _R2, iota, _N)
    cols = []
    for _ in range(_S):
        m = jnp.min(cand, axis=1, keepdims=True)
        cols.append(m)
        cand = jnp.where(cand == m, _N, cand)
    idxt = jnp.concatenate(cols, axis=1)
    idxt = jnp.where(idxt == _N, idxt[:, 0:1], idxt)
    idxt = jnp.minimum(idxt, _N - 1)
    idx_ref[0] = idxt + b * _N


def _run_ball_query(xyz, ctr_t):
    return pl.pallas_call(
        _bq_body,
        grid=(_B, _P // 8),
        in_specs=[
            pl.BlockSpec((1, 3, _N), lambda b, t: (b, 0, 0)),
            pl.BlockSpec((1, 8, 3), lambda b, t: (b, t, 0)),
        ],
        out_specs=pl.BlockSpec((1, 8, _S), lambda b, t: (b, t, 0)),
        out_shape=jax.ShapeDtypeStruct((_B, _P, _S), jnp.int32),
    )(xyz, ctr_t)


def _sc_gather_body(tab_ref, idx_ref, out_ref, idx_v, rows_v, sem):
    wid = lax.axis_index("s") * 2 + lax.axis_index("c")
    per_w = _TOT // _NW
    base = wid * per_w

    def chunk(c, carry):
        off = base + c * _CHUNK
        pltpu.sync_copy(idx_ref.at[pl.ds(off, _CHUNK)], idx_v)
        pltpu.async_copy(tab_ref.at[idx_v], rows_v, sem).wait()
        pltpu.sync_copy(rows_v, out_ref.at[pl.ds(off, _CHUNK)])
        return carry

    lax.fori_loop(0, per_w // _CHUNK, chunk, 0)


def _sc_gather(table, flat_idx):
    mesh = plsc.VectorSubcoreMesh(core_axis_name="c", subcore_axis_name="s")
    f = functools.partial(
        pl.kernel,
        mesh=mesh,
        out_type=jax.ShapeDtypeStruct((_TOT, 128), jnp.float32),
        scratch_types=[
            pltpu.VMEM((_CHUNK,), jnp.int32),
            pltpu.VMEM((_CHUNK, 128), jnp.float32),
            pltpu.SemaphoreType.DMA,
        ],
    )(_sc_gather_body)
    return f(table, flat_idx)


def _mlp1_body(g_ref, c_ref, w_ref, b_ref, y_ref, st_ref, acc):
    i = pl.program_id(0)

    @pl.when(i == 0)
    def _():
        acc[...] = jnp.zeros_like(acc)

    g = g_ref[...][:, 0:_D]
    c = c_ref[...]
    ce = jnp.reshape(jnp.broadcast_to(c[:, None, :], (_ROWS // _S, _S, 3)),
                     (_ROWS, 3))
    xcat = jnp.concatenate([g[:, 0:3] - ce, g[:, 3:]], axis=1)
    yv = jnp.dot(xcat.astype(jnp.bfloat16), w_ref[...].astype(jnp.bfloat16),
                 preferred_element_type=jnp.float32) + b_ref[...]
    y_ref[...] = yv
    acc[0:1, :] = acc[0:1, :] + jnp.sum(yv, axis=0, keepdims=True)
    acc[1:2, :] = acc[1:2, :] + jnp.sum(yv * yv, axis=0, keepdims=True)

    @pl.when(i == _GRID1 - 1)
    def _():
        st_ref[...] = acc[...]


def _run_mlp1(g, ctr_rows, w0p, b0):
    return pl.pallas_call(
        _mlp1_body,
        grid=(_GRID1,),
        in_specs=[
            pl.BlockSpec((_ROWS, 128), lambda i: (i, 0)),
            pl.BlockSpec((_ROWS // _S, 3), lambda i: (i, 0)),
            pl.BlockSpec((_D, _D), lambda i: (0, 0)),
            pl.BlockSpec((1, _D), lambda i: (0, 0)),
        ],
        out_specs=[
            pl.BlockSpec((_ROWS, _D), lambda i: (i, 0)),
            pl.BlockSpec((8, _D), lambda i: (0, 0)),
        ],
        out_shape=[
            jax.ShapeDtypeStruct((_TOT, _D), jnp.float32),
            jax.ShapeDtypeStruct((8, _D), jnp.float32),
        ],
        scratch_shapes=[pltpu.VMEM((8, _D), jnp.float32)],
    )(g, ctr_rows, w0p, b0)


def _mlp2_body(y_ref, a_ref, d_ref, w_ref, b_ref, o_ref, st_ref, acc):
    i = pl.program_id(0)

    @pl.when(i == 0)
    def _():
        acc[...] = jnp.zeros_like(acc)

    z = jnp.maximum(y_ref[...] * a_ref[...] + d_ref[...], 0.0)
    yv = jnp.dot(z.astype(jnp.bfloat16), w_ref[...].astype(jnp.bfloat16),
                 preferred_element_type=jnp.float32) + b_ref[...]
    o_ref[...] = yv
    acc[0:1, :] = acc[0:1, :] + jnp.sum(yv, axis=0, keepdims=True)
    acc[1:2, :] = acc[1:2, :] + jnp.sum(yv * yv, axis=0, keepdims=True)

    @pl.when(i == _GRID1 - 1)
    def _():
        st_ref[...] = acc[...]


def _run_mlp2(y0, a0, d0, w1p, b1):
    return pl.pallas_call(
        _mlp2_body,
        grid=(_GRID1,),
        in_specs=[
            pl.BlockSpec((_ROWS, _D), lambda i: (i, 0)),
            pl.BlockSpec((1, _D), lambda i: (0, 0)),
            pl.BlockSpec((1, _D), lambda i: (0, 0)),
            pl.BlockSpec((_D, _D), lambda i: (0, 0)),
            pl.BlockSpec((1, _D), lambda i: (0, 0)),
        ],
        out_specs=[
            pl.BlockSpec((_ROWS, _D), lambda i: (i, 0)),
            pl.BlockSpec((8, _D), lambda i: (0, 0)),
        ],
        out_shape=[
            jax.ShapeDtypeStruct((_TOT, _D), jnp.float32),
            jax.ShapeDtypeStruct((8, _D), jnp.float32),
        ],
        scratch_shapes=[pltpu.VMEM((8, _D), jnp.float32)],
    )(y0, a0, d0, w1p, b1)


def _mlp3_body(y_ref, a_ref, d_ref, w_ref, b_ref, mx_ref, mn_ref, st_ref, acc):
    i = pl.program_id(0)

    @pl.when(i == 0)
    def _():
        acc[...] = jnp.zeros_like(acc)

    z = jnp.maximum(y_ref[...] * a_ref[...] + d_ref[...], 0.0)
    yv = jnp.dot(z.astype(jnp.bfloat16), w_ref[...].astype(jnp.bfloat16),
                 preferred_element_type=jnp.float32) + b_ref[...]
    acc[0:1, :] = acc[0:1, :] + jnp.sum(yv, axis=0, keepdims=True)
    acc[1:2, :] = acc[1:2, :] + jnp.sum(yv * yv, axis=0, keepdims=True)
    yr = yv.reshape(_ROWS // _S, _S, 64)
    mx_ref[...] = jnp.max(yr, axis=1)
    mn_ref[...] = jnp.min(yr, axis=1)

    @pl.when(i == _GRID1 - 1)
    def _():
        st_ref[...] = acc[...]


def _run_mlp3(y1, a1, d1, w2p, b2):
    return pl.pallas_call(
        _mlp3_body,
        grid=(_GRID1,),
        in_specs=[
            pl.BlockSpec((_ROWS, _D), lambda i: (i, 0)),
            pl.BlockSpec((1, _D), lambda i: (0, 0)),
            pl.BlockSpec((1, _D), lambda i: (0, 0)),
            pl.BlockSpec((_D, 64), lambda i: (0, 0)),
            pl.BlockSpec((1, 64), lambda i: (0, 0)),
        ],
        out_specs=[
            pl.BlockSpec((_ROWS // _S, 64), lambda i: (i, 0)),
            pl.BlockSpec((_ROWS // _S, 64), lambda i: (i, 0)),
            pl.BlockSpec((8, 64), lambda i: (0, 0)),
        ],
        out_shape=[
            jax.ShapeDtypeStruct((_B * _P, 64), jnp.float32),
            jax.ShapeDtypeStruct((_B * _P, 64), jnp.float32),
            jax.ShapeDtypeStruct((8, 64), jnp.float32),
        ],
        scratch_shapes=[pltpu.VMEM((8, 64), jnp.float32)],
    )(y1, a1, d1, w2p, b2)


def _fin_body(mx_ref, mn_ref, g_ref, a_ref, d_ref, o_ref):
    g = g_ref[...]
    sel = jnp.where(g >= 0.0, mx_ref[...], mn_ref[...])
    o_ref[...] = jnp.maximum(sel * a_ref[...] + d_ref[...], 0.0)


def _run_final(mx, mn, g2, a2, d2):
    return pl.pallas_call(
        _fin_body,
        grid=(16,),
        in_specs=[
            pl.BlockSpec((_ROWS, 64), lambda i: (i, 0)),
            pl.BlockSpec((_ROWS, 64), lambda i: (i, 0)),
            pl.BlockSpec((1, 64), lambda i: (0, 0)),
            pl.BlockSpec((1, 64), lambda i: (0, 0)),
            pl.BlockSpec((1, 64), lambda i: (0, 0)),
        ],
        out_specs=pl.BlockSpec((_ROWS, 64), lambda i: (i, 0)),
        out_shape=jax.ShapeDtypeStruct((_B * _P, 64), jnp.float32),
    )(mx, mn, g2, a2, d2)


def _bn_affine(s, ss, gamma, beta):
    cnt = float(_TOT)
    mean = s / cnt
    var = jnp.maximum(ss / cnt - mean * mean, 0.0)
    a = gamma / jnp.sqrt(var + _EPS)
    d = beta - mean * a
    return a, d


def kernel(xyz, points, w0, b0, gamma0, beta0, w1, b1, gamma1, beta1,
           w2, b2, gamma2, beta2):
    new_xyz = _run_fps(xyz)
    ctr_t = jnp.transpose(new_xyz, (0, 2, 1))

    idx = _run_ball_query(xyz, ctr_t)

    table = jnp.concatenate(
        [jnp.transpose(xyz, (0, 2, 1)),
         jnp.transpose(points, (0, 2, 1)),
         jnp.zeros((_B, _N, 128 - 19), jnp.float32)], axis=-1
    ).reshape(_B * _N, 128)
    g = _sc_gather(table, idx.reshape(_TOT))

    w0p = jnp.zeros((_D, _D), jnp.float32).at[:19, :].set(w0.T)
    w1p = w1.T
    w2p = w2.T
    ctr_rows = ctr_t.reshape(_B * _P, 3)
    y0, st0 = _run_mlp1(g, ctr_rows, w0p, b0.reshape(1, _D))
    a0, d0 = _bn_affine(st0[0], st0[1], gamma0, beta0)
    y1, st1 = _run_mlp2(y0, a0.reshape(1, _D), d0.reshape(1, _D),
                        w1p, b1.reshape(1, _D))
    a1, d1 = _bn_affine(st1[0], st1[1], gamma1, beta1)
    mx, mn, st2 = _run_mlp3(y1, a1.reshape(1, _D), d1.reshape(1, _D),
                            w2p, b2.reshape(1, 64))
    a2, d2 = _bn_affine(st2[0, :], st2[1, :], gamma2, beta2)
    pts = _run_final(mx, mn, gamma2.reshape(1, 64), a2.reshape(1, 64),
                     d2.reshape(1, 64))
    new_points = jnp.transpose(pts.reshape(_B, _P, 64), (0, 2, 1))
    return new_xyz, new_points


# speedup vs baseline: 5.4219x; 1.0012x over previous
import functools

import jax
import jax.numpy as jnp
from jax import lax
from jax.experimental import pallas as pl
from jax.experimental.pallas import tpu as pltpu
from jax.experimental.pallas import tpu_sc as plsc

_B, _N, _P, _S = 8, 8192, 1024, 32
_R2 = 0.4 * 0.4
_EPS = 1e-5
_NSUB, _NLANE = 8, 1024
_TOT = _B * _P * _S
_D = 32
_ROWS = 512
_GRID1 = _TOT // _ROWS
_NW = 32
_CHUNK = 128


def _fps_body(xyz_ref, ctr_ref):
    x = xyz_ref[0]
    y = xyz_ref[1]
    z = xyz_ref[2]
    iota_n = lax.broadcasted_iota(jnp.int32, (_B, _N), 1)
    iota_p = lax.broadcasted_iota(jnp.int32, (_B, _P), 1)

    def body(i, st):
        dists, far, ax, ay, az = st
        oh = iota_n == far
        cx = jnp.sum(jnp.where(oh, x, 0.0), axis=1, keepdims=True)
        cy = jnp.sum(jnp.where(oh, y, 0.0), axis=1, keepdims=True)
        cz = jnp.sum(jnp.where(oh, z, 0.0), axis=1, keepdims=True)
        rec = iota_p == i
        ax = jnp.where(rec, cx, ax)
        ay = jnp.where(rec, cy, ay)
        az = jnp.where(rec, cz, az)
        dx = x - cx
        dy = y - cy
        dz = z - cz
        d = dx * dx + dy * dy + dz * dz
        dists = jnp.minimum(dists, d)
        m = jnp.max(dists, axis=1, keepdims=True)
        far = jnp.min(jnp.where(dists == m, iota_n, _N), axis=1,
                      keepdims=True).astype(jnp.int32)
        return dists, far, ax, ay, az

    init = (jnp.full((_B, _N), 1e10, jnp.float32),
            jnp.zeros((_B, 1), jnp.int32),
            jnp.zeros((_B, _P), jnp.float32),
            jnp.zeros((_B, _P), jnp.float32),
            jnp.zeros((_B, _P), jnp.float32))
    _, _, ax, ay, az = lax.fori_loop(0, _P, body, init)
    ctr_ref[0] = ax
    ctr_ref[1] = ay
    ctr_ref[2] = az


def _run_fps(xyz):
    xyz_sb = jnp.transpose(xyz, (1, 0, 2))
    ctr = pl.pallas_call(
        _fps_body,
        grid=(1,),
        in_specs=[pl.BlockSpec((3, _B, _N), lambda i: (0, 0, 0))],
        out_specs=pl.BlockSpec((3, _B, _P), lambda i: (0, 0, 0)),
        out_shape=jax.ShapeDtypeStruct((3, _B, _P), jnp.float32),
    )(xyz_sb)
    return jnp.transpose(ctr, (1, 0, 2))


def _bq_body(xyz_ref, ctr_ref, idx_ref):
    b = pl.program_id(0)
    x = xyz_ref[0, 0:1, :]
    y = xyz_ref[0, 1:2, :]
    z = xyz_ref[0, 2:3, :]
    cx = ctr_ref[0, :, 0:1]
    cy = ctr_ref[0, :, 1:2]
    cz = ctr_ref[0, :, 2:3]
    xx = x * x + y * y + z * z
    cc = cx * cx + cy * cy + cz * cz
    xb = x.astype(jnp.bfloat16).astype(jnp.float32)
    yb = y.astype(jnp.bfloat16).astype(jnp.float32)
    zb = z.astype(jnp.bfloat16).astype(jnp.float32)
    cxb = cx.astype(jnp.bfloat16).astype(jnp.float32)
    cyb = cy.astype(jnp.bfloat16).astype(jnp.float32)
    czb = cz.astype(jnp.bfloat16).astype(jnp.float32)
    cross = cxb * xb + cyb * yb + czb * zb
    d2 = (cc + xx) - 2.0 * cross
    iota = lax.broadcasted_iota(jnp.int32, (8, _N), 1)
    cand = jnp.where(d2 <= _R2, iota, _N)
    m = jnp.min(cand, axis=1, keepdims=True)
    cols = [m]
    for _ in range(_S - 1):
        m = jnp.min(jnp.where(cand > m, cand, _N), axis=1, keepdims=True)
        cols.append(m)
    idxt = jnp.concatenate(cols, axis=1)
    idxt = jnp.where(idxt == _N, idxt[:, 0:1], idxt)
    idxt = jnp.minimum(idxt, _N - 1)
    idx_ref[0] = idxt + b * _N


def _run_ball_query(xyz, ctr_t):
    return pl.pallas_call(
        _bq_body,
        grid=(_B, _P // 8),
        in_specs=[
            pl.BlockSpec((1, 3, _N), lambda b, t: (b, 0, 0)),
            pl.BlockSpec((1, 8, 3), lambda b, t: (b, t, 0)),
        ],
        out_specs=pl.BlockSpec((1, 8, _S), lambda b, t: (b, t, 0)),
        out_shape=jax.ShapeDtypeStruct((_B, _P, _S), jnp.int32),
    )(xyz, ctr_t)


def _sc_gather_body(tab_ref, idx_ref, out_ref, idx_v, rows_v, sem):
    wid = lax.axis_index("s") * 2 + lax.axis_index("c")
    per_w = _TOT // _NW
    base = wid * per_w

    def chunk(c, carry):
        off = base + c * _CHUNK
        pltpu.sync_copy(idx_ref.at[pl.ds(off, _CHUNK)], idx_v)
        pltpu.async_copy(tab_ref.at[idx_v], rows_v, sem).wait()
        pltpu.sync_copy(rows_v, out_ref.at[pl.ds(off, _CHUNK)])
        return carry

    lax.fori_loop(0, per_w // _CHUNK, chunk, 0)


def _sc_gather(table, flat_idx):
    mesh = plsc.VectorSubcoreMesh(core_axis_name="c", subcore_axis_name="s")
    f = functools.partial(
        pl.kernel,
        mesh=mesh,
        out_type=jax.ShapeDtypeStruct((_TOT, 128), jnp.float32),
        scratch_types=[
            pltpu.VMEM((_CHUNK,), jnp.int32),
            pltpu.VMEM((_CHUNK, 128), jnp.float32),
            pltpu.SemaphoreType.DMA,
        ],
    )(_sc_gather_body)
    return f(table, flat_idx)


def _mlp1_body(g_ref, c_ref, w_ref, b_ref, y_ref, st_ref, acc):
    i = pl.program_id(0)

    @pl.when(i == 0)
    def _():
        acc[...] = jnp.zeros_like(acc)

    g = g_ref[...][:, 0:_D]
    c = c_ref[...]
    ce = jnp.reshape(jnp.broadcast_to(c[:, None, :], (_ROWS // _S, _S, 3)),
                     (_ROWS, 3))
    xcat = jnp.concatenate([g[:, 0:3] - ce, g[:, 3:]], axis=1)
    yv = jnp.dot(xcat.astype(jnp.bfloat16), w_ref[...].astype(jnp.bfloat16),
                 preferred_element_type=jnp.float32) + b_ref[...]
    y_ref[...] = yv
    acc[0:1, :] = acc[0:1, :] + jnp.sum(yv, axis=0, keepdims=True)
    acc[1:2, :] = acc[1:2, :] + jnp.sum(yv * yv, axis=0, keepdims=True)

    @pl.when(i == _GRID1 - 1)
    def _():
        st_ref[...] = acc[...]


def _run_mlp1(g, ctr_rows, w0p, b0):
    return pl.pallas_call(
        _mlp1_body,
        grid=(_GRID1,),
        in_specs=[
            pl.BlockSpec((_ROWS, 128), lambda i: (i, 0)),
            pl.BlockSpec((_ROWS // _S, 3), lambda i: (i, 0)),
            pl.BlockSpec((_D, _D), lambda i: (0, 0)),
            pl.BlockSpec((1, _D), lambda i: (0, 0)),
        ],
        out_specs=[
            pl.BlockSpec((_ROWS, _D), lambda i: (i, 0)),
            pl.BlockSpec((8, _D), lambda i: (0, 0)),
        ],
        out_shape=[
            jax.ShapeDtypeStruct((_TOT, _D), jnp.float32),
            jax.ShapeDtypeStruct((8, _D), jnp.float32),
        ],
        scratch_shapes=[pltpu.VMEM((8, _D), jnp.float32)],
    )(g, ctr_rows, w0p, b0)


def _mlp2_body(y_ref, a_ref, d_ref, w_ref, b_ref, o_ref, st_ref, acc):
    i = pl.program_id(0)

    @pl.when(i == 0)
    def _():
        acc[...] = jnp.zeros_like(acc)

    z = jnp.maximum(y_ref[...] * a_ref[...] + d_ref[...], 0.0)
    yv = jnp.dot(z.astype(jnp.bfloat16), w_ref[...].astype(jnp.bfloat16),
                 preferred_element_type=jnp.float32) + b_ref[...]
    o_ref[...] = yv
    acc[0:1, :] = acc[0:1, :] + jnp.sum(yv, axis=0, keepdims=True)
    acc[1:2, :] = acc[1:2, :] + jnp.sum(yv * yv, axis=0, keepdims=True)

    @pl.when(i == _GRID1 - 1)
    def _():
        st_ref[...] = acc[...]


def _run_mlp2(y0, a0, d0, w1p, b1):
    return pl.pallas_call(
        _mlp2_body,
        grid=(_GRID1,),
        in_specs=[
            pl.BlockSpec((_ROWS, _D), lambda i: (i, 0)),
            pl.BlockSpec((1, _D), lambda i: (0, 0)),
            pl.BlockSpec((1, _D), lambda i: (0, 0)),
            pl.BlockSpec((_D, _D), lambda i: (0, 0)),
            pl.BlockSpec((1, _D), lambda i: (0, 0)),
        ],
        out_specs=[
            pl.BlockSpec((_ROWS, _D), lambda i: (i, 0)),
            pl.BlockSpec((8, _D), lambda i: (0, 0)),
        ],
        out_shape=[
            jax.ShapeDtypeStruct((_TOT, _D), jnp.float32),
            jax.ShapeDtypeStruct((8, _D), jnp.float32),
        ],
        scratch_shapes=[pltpu.VMEM((8, _D), jnp.float32)],
    )(y0, a0, d0, w1p, b1)


def _mlp3_body(y_ref, a_ref, d_ref, w_ref, b_ref, mx_ref, mn_ref, st_ref, acc):
    i = pl.program_id(0)

    @pl.when(i == 0)
    def _():
        acc[...] = jnp.zeros_like(acc)

    z = jnp.maximum(y_ref[...] * a_ref[...] + d_ref[...], 0.0)
    yv = jnp.dot(z.astype(jnp.bfloat16), w_ref[...].astype(jnp.bfloat16),
                 preferred_element_type=jnp.float32) + b_ref[...]
    acc[0:1, :] = acc[0:1, :] + jnp.sum(yv, axis=0, keepdims=True)
    acc[1:2, :] = acc[1:2, :] + jnp.sum(yv * yv, axis=0, keepdims=True)
    yr = yv.reshape(_ROWS // _S, _S, 64)
    mx_ref[...] = jnp.max(yr, axis=1)
    mn_ref[...] = jnp.min(yr, axis=1)

    @pl.when(i == _GRID1 - 1)
    def _():
        st_ref[...] = acc[...]


def _run_mlp3(y1, a1, d1, w2p, b2):
    return pl.pallas_call(
        _mlp3_body,
        grid=(_GRID1,),
        in_specs=[
            pl.BlockSpec((_ROWS, _D), lambda i: (i, 0)),
            pl.BlockSpec((1, _D), lambda i: (0, 0)),
            pl.BlockSpec((1, _D), lambda i: (0, 0)),
            pl.BlockSpec((_D, 64), lambda i: (0, 0)),
            pl.BlockSpec((1, 64), lambda i: (0, 0)),
        ],
        out_specs=[
            pl.BlockSpec((_ROWS // _S, 64), lambda i: (i, 0)),
            pl.BlockSpec((_ROWS // _S, 64), lambda i: (i, 0)),
            pl.BlockSpec((8, 64), lambda i: (0, 0)),
        ],
        out_shape=[
            jax.ShapeDtypeStruct((_B * _P, 64), jnp.float32),
            jax.ShapeDtypeStruct((_B * _P, 64), jnp.float32),
            jax.ShapeDtypeStruct((8, 64), jnp.float32),
        ],
        scratch_shapes=[pltpu.VMEM((8, 64), jnp.float32)],
    )(y1, a1, d1, w2p, b2)


def _fin_body(mx_ref, mn_ref, g_ref, a_ref, d_ref, o_ref):
    g = g_ref[...]
    sel = jnp.where(g >= 0.0, mx_ref[...], mn_ref[...])
    o_ref[...] = jnp.maximum(sel * a_ref[...] + d_ref[...], 0.0)


def _run_final(mx, mn, g2, a2, d2):
    return pl.pallas_call(
        _fin_body,
        grid=(16,),
        in_specs=[
            pl.BlockSpec((_ROWS, 64), lambda i: (i, 0)),
            pl.BlockSpec((_ROWS, 64), lambda i: (i, 0)),
            pl.BlockSpec((1, 64), lambda i: (0, 0)),
            pl.BlockSpec((1, 64), lambda i: (0, 0)),
            pl.BlockSpec((1, 64), lambda i: (0, 0)),
        ],
        out_specs=pl.BlockSpec((_ROWS, 64), lambda i: (i, 0)),
        out_shape=jax.ShapeDtypeStruct((_B * _P, 64), jnp.float32),
    )(mx, mn, g2, a2, d2)


def _bn_affine(s, ss, gamma, beta):
    cnt = float(_TOT)
    mean = s / cnt
    var = jnp.maximum(ss / cnt - mean * mean, 0.0)
    a = gamma / jnp.sqrt(var + _EPS)
    d = beta - mean * a
    return a, d


def kernel(xyz, points, w0, b0, gamma0, beta0, w1, b1, gamma1, beta1,
           w2, b2, gamma2, beta2):
    new_xyz = _run_fps(xyz)
    ctr_t = jnp.transpose(new_xyz, (0, 2, 1))

    idx = _run_ball_query(xyz, ctr_t)

    table = jnp.concatenate(
        [jnp.transpose(xyz, (0, 2, 1)),
         jnp.transpose(points, (0, 2, 1)),
         jnp.zeros((_B, _N, 128 - 19), jnp.float32)], axis=-1
    ).reshape(_B * _N, 128)
    g = _sc_gather(table, idx.reshape(_TOT))

    w0p = jnp.zeros((_D, _D), jnp.float32).at[:19, :].set(w0.T)
    w1p = w1.T
    w2p = w2.T
    ctr_rows = ctr_t.reshape(_B * _P, 3)
    y0, st0 = _run_mlp1(g, ctr_rows, w0p, b0.reshape(1, _D))
    a0, d0 = _bn_affine(st0[0], st0[1], gamma0, beta0)
    y1, st1 = _run_mlp2(y0, a0.reshape(1, _D), d0.reshape(1, _D),
                        w1p, b1.reshape(1, _D))
    a1, d1 = _bn_affine(st1[0], st1[1], gamma1, beta1)
    mx, mn, st2 = _run_mlp3(y1, a1.reshape(1, _D), d1.reshape(1, _D),
                            w2p, b2.reshape(1, 64))
    a2, d2 = _bn_affine(st2[0, :], st2[1, :], gamma2, beta2)
    pts = _run_final(mx, mn, gamma2.reshape(1, 64), a2.reshape(1, 64),
                     d2.reshape(1, 64))
    new_points = jnp.transpose(pts.reshape(_B, _P, 64), (0, 2, 1))
    return new_xyz, new_points


# speedup vs baseline: 11.3813x; 2.0991x over previous
import functools

import jax
import jax.numpy as jnp
from jax import lax
from jax.experimental import pallas as pl
from jax.experimental.pallas import tpu as pltpu
from jax.experimental.pallas import tpu_sc as plsc

_B, _N, _P, _S = 8, 8192, 1024, 32
_R2 = 0.4 * 0.4
_EPS = 1e-5
_NSUB, _NLANE = 8, 1024
_TOT = _B * _P * _S
_D = 32
_ROWS = 512
_GRID1 = _TOT // _ROWS
_NW = 32
_CHUNK = 128


def _fps_body(xyz_ref, ctr_ref):
    x = xyz_ref[0]
    y = xyz_ref[1]
    z = xyz_ref[2]
    iota_n = lax.broadcasted_iota(jnp.int32, (_B, _N), 1)
    iota_p = lax.broadcasted_iota(jnp.int32, (_B, _P), 1)

    def body(i, st):
        dists, far, ax, ay, az = st
        oh = iota_n == far
        cx = jnp.sum(jnp.where(oh, x, 0.0), axis=1, keepdims=True)
        cy = jnp.sum(jnp.where(oh, y, 0.0), axis=1, keepdims=True)
        cz = jnp.sum(jnp.where(oh, z, 0.0), axis=1, keepdims=True)
        rec = iota_p == i
        ax = jnp.where(rec, cx, ax)
        ay = jnp.where(rec, cy, ay)
        az = jnp.where(rec, cz, az)
        dx = x - cx
        dy = y - cy
        dz = z - cz
        d = dx * dx + dy * dy + dz * dz
        dists = jnp.minimum(dists, d)
        m = jnp.max(dists, axis=1, keepdims=True)
        far = jnp.min(jnp.where(dists == m, iota_n, _N), axis=1,
                      keepdims=True).astype(jnp.int32)
        return dists, far, ax, ay, az

    init = (jnp.full((_B, _N), 1e10, jnp.float32),
            jnp.zeros((_B, 1), jnp.int32),
            jnp.zeros((_B, _P), jnp.float32),
            jnp.zeros((_B, _P), jnp.float32),
            jnp.zeros((_B, _P), jnp.float32))
    _, _, ax, ay, az = lax.fori_loop(0, _P, body, init)
    ctr_ref[0] = ax
    ctr_ref[1] = ay
    ctr_ref[2] = az


def _run_fps(xyz):
    xyz_sb = jnp.transpose(xyz, (1, 0, 2))
    ctr = pl.pallas_call(
        _fps_body,
        grid=(1,),
        in_specs=[pl.BlockSpec((3, _B, _N), lambda i: (0, 0, 0))],
        out_specs=pl.BlockSpec((3, _B, _P), lambda i: (0, 0, 0)),
        out_shape=jax.ShapeDtypeStruct((3, _B, _P), jnp.float32),
    )(xyz_sb)
    return jnp.transpose(ctr, (1, 0, 2))


_PT = 32


def _bq_body(xyz_ref, ctr_ref, idx_ref):
    b = pl.program_id(0)
    x = xyz_ref[0, 0:1, :]
    y = xyz_ref[0, 1:2, :]
    z = xyz_ref[0, 2:3, :]
    cx = ctr_ref[0, :, 0:1]
    cy = ctr_ref[0, :, 1:2]
    cz = ctr_ref[0, :, 2:3]
    xx = x * x + y * y + z * z
    cc = cx * cx + cy * cy + cz * cz
    xb = x.astype(jnp.bfloat16).astype(jnp.float32)
    yb = y.astype(jnp.bfloat16).astype(jnp.float32)
    zb = z.astype(jnp.bfloat16).astype(jnp.float32)
    cxb = cx.astype(jnp.bfloat16).astype(jnp.float32)
    cyb = cy.astype(jnp.bfloat16).astype(jnp.float32)
    czb = cz.astype(jnp.bfloat16).astype(jnp.float32)
    cross = cxb * xb + cyb * yb + czb * zb
    d2 = (cc + xx) - 2.0 * cross
    iota = lax.broadcasted_iota(jnp.int32, (_PT, _N), 1)
    cand = jnp.where(d2 <= _R2, iota, _N)

    def _rowmin(v):
        parts = [v[:, k * 512:(k + 1) * 512] for k in range(v.shape[1] // 512)]
        while len(parts) > 1:
            parts = [jnp.minimum(parts[i], parts[i + 1])
                     for i in range(0, len(parts), 2)]
        return jnp.min(parts[0], axis=1, keepdims=True)

    m = _rowmin(cand)
    cols = [m]
    for _ in range(_S - 1):
        m = _rowmin(jnp.where(cand > m, cand, _N))
        cols.append(m)
    idxt = jnp.concatenate(cols, axis=1)
    idxt = jnp.where(idxt == _N, idxt[:, 0:1], idxt)
    idxt = jnp.minimum(idxt, _N - 1)
    idx_ref[0] = idxt + b * _N


def _run_ball_query(xyz, ctr_t):
    return pl.pallas_call(
        _bq_body,
        grid=(_B, _P // _PT),
        in_specs=[
            pl.BlockSpec((1, 3, _N), lambda b, t: (b, 0, 0)),
            pl.BlockSpec((1, _PT, 3), lambda b, t: (b, t, 0)),
        ],
        out_specs=pl.BlockSpec((1, _PT, _S), lambda b, t: (b, t, 0)),
        out_shape=jax.ShapeDtypeStruct((_B, _P, _S), jnp.int32),
    )(xyz, ctr_t)


def _sc_gather_body(tab_ref, idx_ref, out_ref, idx_v, rows_v, sem):
    wid = lax.axis_index("s") * 2 + lax.axis_index("c")
    per_w = _TOT // _NW
    base = wid * per_w

    def chunk(c, carry):
        off = base + c * _CHUNK
        pltpu.sync_copy(idx_ref.at[pl.ds(off, _CHUNK)], idx_v)
        pltpu.async_copy(tab_ref.at[idx_v], rows_v, sem).wait()
        pltpu.sync_copy(rows_v, out_ref.at[pl.ds(off, _CHUNK)])
        return carry

    lax.fori_loop(0, per_w // _CHUNK, chunk, 0)


def _sc_gather(table, flat_idx):
    mesh = plsc.VectorSubcoreMesh(core_axis_name="c", subcore_axis_name="s")
    f = functools.partial(
        pl.kernel,
        mesh=mesh,
        out_type=jax.ShapeDtypeStruct((_TOT, 128), jnp.float32),
        scratch_types=[
            pltpu.VMEM((_CHUNK,), jnp.int32),
            pltpu.VMEM((_CHUNK, 128), jnp.float32),
            pltpu.SemaphoreType.DMA,
        ],
    )(_sc_gather_body)
    return f(table, flat_idx)


def _mlp1_body(g_ref, c_ref, w_ref, b_ref, y_ref, st_ref, acc):
    i = pl.program_id(0)

    @pl.when(i == 0)
    def _():
        acc[...] = jnp.zeros_like(acc)

    g = g_ref[...][:, 0:_D]
    c = c_ref[...]
    ce = jnp.reshape(jnp.broadcast_to(c[:, None, :], (_ROWS // _S, _S, 3)),
                     (_ROWS, 3))
    xcat = jnp.concatenate([g[:, 0:3] - ce, g[:, 3:]], axis=1)
    yv = jnp.dot(xcat.astype(jnp.bfloat16), w_ref[...].astype(jnp.bfloat16),
                 preferred_element_type=jnp.float32) + b_ref[...]
    y_ref[...] = yv
    acc[0:1, :] = acc[0:1, :] + jnp.sum(yv, axis=0, keepdims=True)
    acc[1:2, :] = acc[1:2, :] + jnp.sum(yv * yv, axis=0, keepdims=True)

    @pl.when(i == _GRID1 - 1)
    def _():
        st_ref[...] = acc[...]


def _run_mlp1(g, ctr_rows, w0p, b0):
    return pl.pallas_call(
        _mlp1_body,
        grid=(_GRID1,),
        in_specs=[
            pl.BlockSpec((_ROWS, 128), lambda i: (i, 0)),
            pl.BlockSpec((_ROWS // _S, 3), lambda i: (i, 0)),
            pl.BlockSpec((_D, _D), lambda i: (0, 0)),
            pl.BlockSpec((1, _D), lambda i: (0, 0)),
        ],
        out_specs=[
            pl.BlockSpec((_ROWS, _D), lambda i: (i, 0)),
            pl.BlockSpec((8, _D), lambda i: (0, 0)),
        ],
        out_shape=[
            jax.ShapeDtypeStruct((_TOT, _D), jnp.float32),
            jax.ShapeDtypeStruct((8, _D), jnp.float32),
        ],
        scratch_shapes=[pltpu.VMEM((8, _D), jnp.float32)],
    )(g, ctr_rows, w0p, b0)


def _mlp2_body(y_ref, a_ref, d_ref, w_ref, b_ref, o_ref, st_ref, acc):
    i = pl.program_id(0)

    @pl.when(i == 0)
    def _():
        acc[...] = jnp.zeros_like(acc)

    z = jnp.maximum(y_ref[...] * a_ref[...] + d_ref[...], 0.0)
    yv = jnp.dot(z.astype(jnp.bfloat16), w_ref[...].astype(jnp.bfloat16),
                 preferred_element_type=jnp.float32) + b_ref[...]
    o_ref[...] = yv
    acc[0:1, :] = acc[0:1, :] + jnp.sum(yv, axis=0, keepdims=True)
    acc[1:2, :] = acc[1:2, :] + jnp.sum(yv * yv, axis=0, keepdims=True)

    @pl.when(i == _GRID1 - 1)
    def _():
        st_ref[...] = acc[...]


def _run_mlp2(y0, a0, d0, w1p, b1):
    return pl.pallas_call(
        _mlp2_body,
        grid=(_GRID1,),
        in_specs=[
            pl.BlockSpec((_ROWS, _D), lambda i: (i, 0)),
            pl.BlockSpec((1, _D), lambda i: (0, 0)),
            pl.BlockSpec((1, _D), lambda i: (0, 0)),
            pl.BlockSpec((_D, _D), lambda i: (0, 0)),
            pl.BlockSpec((1, _D), lambda i: (0, 0)),
        ],
        out_specs=[
            pl.BlockSpec((_ROWS, _D), lambda i: (i, 0)),
            pl.BlockSpec((8, _D), lambda i: (0, 0)),
        ],
        out_shape=[
            jax.ShapeDtypeStruct((_TOT, _D), jnp.float32),
            jax.ShapeDtypeStruct((8, _D), jnp.float32),
        ],
        scratch_shapes=[pltpu.VMEM((8, _D), jnp.float32)],
    )(y0, a0, d0, w1p, b1)


def _mlp3_body(y_ref, a_ref, d_ref, w_ref, b_ref, mx_ref, mn_ref, st_ref, acc):
    i = pl.program_id(0)

    @pl.when(i == 0)
    def _():
        acc[...] = jnp.zeros_like(acc)

    z = jnp.maximum(y_ref[...] * a_ref[...] + d_ref[...], 0.0)
    yv = jnp.dot(z.astype(jnp.bfloat16), w_ref[...].astype(jnp.bfloat16),
                 preferred_element_type=jnp.float32) + b_ref[...]
    acc[0:1, :] = acc[0:1, :] + jnp.sum(yv, axis=0, keepdims=True)
    acc[1:2, :] = acc[1:2, :] + jnp.sum(yv * yv, axis=0, keepdims=True)
    yr = yv.reshape(_ROWS // _S, _S, 64)
    mx_ref[...] = jnp.max(yr, axis=1)
    mn_ref[...] = jnp.min(yr, axis=1)

    @pl.when(i == _GRID1 - 1)
    def _():
        st_ref[...] = acc[...]


def _run_mlp3(y1, a1, d1, w2p, b2):
    return pl.pallas_call(
        _mlp3_body,
        grid=(_GRID1,),
        in_specs=[
            pl.BlockSpec((_ROWS, _D), lambda i: (i, 0)),
            pl.BlockSpec((1, _D), lambda i: (0, 0)),
            pl.BlockSpec((1, _D), lambda i: (0, 0)),
            pl.BlockSpec((_D, 64), lambda i: (0, 0)),
            pl.BlockSpec((1, 64), lambda i: (0, 0)),
        ],
        out_specs=[
            pl.BlockSpec((_ROWS // _S, 64), lambda i: (i, 0)),
            pl.BlockSpec((_ROWS // _S, 64), lambda i: (i, 0)),
            pl.BlockSpec((8, 64), lambda i: (0, 0)),
        ],
        out_shape=[
            jax.ShapeDtypeStruct((_B * _P, 64), jnp.float32),
            jax.ShapeDtypeStruct((_B * _P, 64), jnp.float32),
            jax.ShapeDtypeStruct((8, 64), jnp.float32),
        ],
        scratch_shapes=[pltpu.VMEM((8, 64), jnp.float32)],
    )(y1, a1, d1, w2p, b2)


def _fin_body(mx_ref, mn_ref, g_ref, a_ref, d_ref, o_ref):
    g = g_ref[...]
    sel = jnp.where(g >= 0.0, mx_ref[...], mn_ref[...])
    o_ref[...] = jnp.maximum(sel * a_ref[...] + d_ref[...], 0.0)


def _run_final(mx, mn, g2, a2, d2):
    return pl.pallas_call(
        _fin_body,
        grid=(16,),
        in_specs=[
            pl.BlockSpec((_ROWS, 64), lambda i: (i, 0)),
            pl.BlockSpec((_ROWS, 64), lambda i: (i, 0)),
            pl.BlockSpec((1, 64), lambda i: (0, 0)),
            pl.BlockSpec((1, 64), lambda i: (0, 0)),
            pl.BlockSpec((1, 64), lambda i: (0, 0)),
        ],
        out_specs=pl.BlockSpec((_ROWS, 64), lambda i: (i, 0)),
        out_shape=jax.ShapeDtypeStruct((_B * _P, 64), jnp.float32),
    )(mx, mn, g2, a2, d2)


def _bn_affine(s, ss, gamma, beta):
    cnt = float(_TOT)
    mean = s / cnt
    var = jnp.maximum(ss / cnt - mean * mean, 0.0)
    a = gamma / jnp.sqrt(var + _EPS)
    d = beta - mean * a
    return a, d


def kernel(xyz, points, w0, b0, gamma0, beta0, w1, b1, gamma1, beta1,
           w2, b2, gamma2, beta2):
    new_xyz = _run_fps(xyz)
    ctr_t = jnp.transpose(new_xyz, (0, 2, 1))

    idx = _run_ball_query(xyz, ctr_t)

    table = jnp.concatenate(
        [jnp.transpose(xyz, (0, 2, 1)),
         jnp.transpose(points, (0, 2, 1)),
         jnp.zeros((_B, _N, 128 - 19), jnp.float32)], axis=-1
    ).reshape(_B * _N, 128)
    g = _sc_gather(table, idx.reshape(_TOT))

    w0p = jnp.zeros((_D, _D), jnp.float32).at[:19, :].set(w0.T)
    w1p = w1.T
    w2p = w2.T
    ctr_rows = ctr_t.reshape(_B * _P, 3)
    y0, st0 = _run_mlp1(g, ctr_rows, w0p, b0.reshape(1, _D))
    a0, d0 = _bn_affine(st0[0], st0[1], gamma0, beta0)
    y1, st1 = _run_mlp2(y0, a0.reshape(1, _D), d0.reshape(1, _D),
                        w1p, b1.reshape(1, _D))
    a1, d1 = _bn_affine(st1[0], st1[1], gamma1, beta1)
    mx, mn, st2 = _run_mlp3(y1, a1.reshape(1, _D), d1.reshape(1, _D),
                            w2p, b2.reshape(1, 64))
    a2, d2 = _bn_affine(st2[0, :], st2[1, :], gamma2, beta2)
    pts = _run_final(mx, mn, gamma2.reshape(1, 64), a2.reshape(1, 64),
                     d2.reshape(1, 64))
    new_points = jnp.transpose(pts.reshape(_B, _P, 64), (0, 2, 1))
    return new_xyz, new_points


# speedup vs baseline: 12.4076x; 1.0902x over previous
import functools

import jax
import jax.numpy as jnp
from jax import lax
from jax.experimental import pallas as pl
from jax.experimental.pallas import tpu as pltpu
from jax.experimental.pallas import tpu_sc as plsc

_B, _N, _P, _S = 8, 8192, 1024, 32
_R2 = 0.4 * 0.4
_EPS = 1e-5
_NSUB, _NLANE = 8, 1024
_TOT = _B * _P * _S
_D = 32
_ROWS = 512
_GRID1 = _TOT // _ROWS
_NW = 32
_CHUNK = 128


def _fps_body(xyz_ref, ctr_ref):
    x = xyz_ref[0]
    y = xyz_ref[1]
    z = xyz_ref[2]
    iota_n = lax.broadcasted_iota(jnp.int32, (_B, _N), 1)
    iota_p = lax.broadcasted_iota(jnp.int32, (_B, _P), 1)

    def body(i, st):
        dists, far, ax, ay, az = st
        oh = iota_n == far
        cx = jnp.sum(jnp.where(oh, x, 0.0), axis=1, keepdims=True)
        cy = jnp.sum(jnp.where(oh, y, 0.0), axis=1, keepdims=True)
        cz = jnp.sum(jnp.where(oh, z, 0.0), axis=1, keepdims=True)
        rec = iota_p == i
        ax = jnp.where(rec, cx, ax)
        ay = jnp.where(rec, cy, ay)
        az = jnp.where(rec, cz, az)
        dx = x - cx
        dy = y - cy
        dz = z - cz
        d = dx * dx + dy * dy + dz * dz
        dists = jnp.minimum(dists, d)
        m = jnp.max(dists, axis=1, keepdims=True)
        far = jnp.min(jnp.where(dists == m, iota_n, _N), axis=1,
                      keepdims=True).astype(jnp.int32)
        return dists, far, ax, ay, az

    init = (jnp.full((_B, _N), 1e10, jnp.float32),
            jnp.zeros((_B, 1), jnp.int32),
            jnp.zeros((_B, _P), jnp.float32),
            jnp.zeros((_B, _P), jnp.float32),
            jnp.zeros((_B, _P), jnp.float32))
    _, _, ax, ay, az = lax.fori_loop(0, _P, body, init)
    ctr_ref[0] = ax
    ctr_ref[1] = ay
    ctr_ref[2] = az


def _run_fps(xyz):
    xyz_sb = jnp.transpose(xyz, (1, 0, 2))
    ctr = pl.pallas_call(
        _fps_body,
        grid=(1,),
        in_specs=[pl.BlockSpec((3, _B, _N), lambda i: (0, 0, 0))],
        out_specs=pl.BlockSpec((3, _B, _P), lambda i: (0, 0, 0)),
        out_shape=jax.ShapeDtypeStruct((3, _B, _P), jnp.float32),
    )(xyz_sb)
    return jnp.transpose(ctr, (1, 0, 2))


_PT = 64


def _bq_body(xyz_ref, ctr_ref, idx_ref):
    b = pl.program_id(0)
    x = xyz_ref[0, 0:1, :]
    y = xyz_ref[0, 1:2, :]
    z = xyz_ref[0, 2:3, :]
    cx = ctr_ref[0, :, 0:1]
    cy = ctr_ref[0, :, 1:2]
    cz = ctr_ref[0, :, 2:3]
    xx = x * x + y * y + z * z
    cc = cx * cx + cy * cy + cz * cz
    xb = x.astype(jnp.bfloat16).astype(jnp.float32)
    yb = y.astype(jnp.bfloat16).astype(jnp.float32)
    zb = z.astype(jnp.bfloat16).astype(jnp.float32)
    cxb = cx.astype(jnp.bfloat16).astype(jnp.float32)
    cyb = cy.astype(jnp.bfloat16).astype(jnp.float32)
    czb = cz.astype(jnp.bfloat16).astype(jnp.float32)
    cross = cxb * xb + cyb * yb + czb * zb
    d2 = (cc + xx) - 2.0 * cross
    iota = lax.broadcasted_iota(jnp.int32, (_PT, _N), 1)
    cand = jnp.where(d2 <= _R2, iota, _N)

    def _rowmin(v):
        parts = [v[:, k * 512:(k + 1) * 512] for k in range(v.shape[1] // 512)]
        while len(parts) > 1:
            parts = [jnp.minimum(parts[i], parts[i + 1])
                     for i in range(0, len(parts), 2)]
        return jnp.min(parts[0], axis=1, keepdims=True)

    m = _rowmin(cand)
    cols = [m]
    for _ in range(_S - 1):
        m = _rowmin(jnp.where(cand > m, cand, _N))
        cols.append(m)
    idxt = jnp.concatenate(cols, axis=1)
    idxt = jnp.where(idxt == _N, idxt[:, 0:1], idxt)
    idxt = jnp.minimum(idxt, _N - 1)
    idx_ref[0] = idxt + b * _N


def _run_ball_query(xyz, ctr_t):
    return pl.pallas_call(
        _bq_body,
        grid=(_B, _P // _PT),
        in_specs=[
            pl.BlockSpec((1, 3, _N), lambda b, t: (b, 0, 0)),
            pl.BlockSpec((1, _PT, 3), lambda b, t: (b, t, 0)),
        ],
        out_specs=pl.BlockSpec((1, _PT, _S), lambda b, t: (b, t, 0)),
        out_shape=jax.ShapeDtypeStruct((_B, _P, _S), jnp.int32),
    )(xyz, ctr_t)


def _sc_gather_body(tab_ref, idx_ref, out_ref, idx_v, rows_v, sem):
    wid = lax.axis_index("s") * 2 + lax.axis_index("c")
    per_w = _TOT // _NW
    base = wid * per_w

    def chunk(c, carry):
        off = base + c * _CHUNK
        pltpu.sync_copy(idx_ref.at[pl.ds(off, _CHUNK)], idx_v)
        pltpu.async_copy(tab_ref.at[idx_v], rows_v, sem).wait()
        pltpu.sync_copy(rows_v, out_ref.at[pl.ds(off, _CHUNK)])
        return carry

    lax.fori_loop(0, per_w // _CHUNK, chunk, 0)


def _sc_gather(table, flat_idx):
    mesh = plsc.VectorSubcoreMesh(core_axis_name="c", subcore_axis_name="s")
    f = functools.partial(
        pl.kernel,
        mesh=mesh,
        out_type=jax.ShapeDtypeStruct((_TOT, 128), jnp.float32),
        scratch_types=[
            pltpu.VMEM((_CHUNK,), jnp.int32),
            pltpu.VMEM((_CHUNK, 128), jnp.float32),
            pltpu.SemaphoreType.DMA,
        ],
    )(_sc_gather_body)
    return f(table, flat_idx)


def _mlp1_body(g_ref, c_ref, w_ref, b_ref, y_ref, st_ref, acc):
    i = pl.program_id(0)

    @pl.when(i == 0)
    def _():
        acc[...] = jnp.zeros_like(acc)

    g = g_ref[...][:, 0:_D]
    c = c_ref[...]
    ce = jnp.reshape(jnp.broadcast_to(c[:, None, :], (_ROWS // _S, _S, 3)),
                     (_ROWS, 3))
    xcat = jnp.concatenate([g[:, 0:3] - ce, g[:, 3:]], axis=1)
    yv = jnp.dot(xcat.astype(jnp.bfloat16), w_ref[...].astype(jnp.bfloat16),
                 preferred_element_type=jnp.float32) + b_ref[...]
    y_ref[...] = yv
    acc[0:1, :] = acc[0:1, :] + jnp.sum(yv, axis=0, keepdims=True)
    acc[1:2, :] = acc[1:2, :] + jnp.sum(yv * yv, axis=0, keepdims=True)

    @pl.when(i == _GRID1 - 1)
    def _():
        st_ref[...] = acc[...]


def _run_mlp1(g, ctr_rows, w0p, b0):
    return pl.pallas_call(
        _mlp1_body,
        grid=(_GRID1,),
        in_specs=[
            pl.BlockSpec((_ROWS, 128), lambda i: (i, 0)),
            pl.BlockSpec((_ROWS // _S, 3), lambda i: (i, 0)),
            pl.BlockSpec((_D, _D), lambda i: (0, 0)),
            pl.BlockSpec((1, _D), lambda i: (0, 0)),
        ],
        out_specs=[
            pl.BlockSpec((_ROWS, _D), lambda i: (i, 0)),
            pl.BlockSpec((8, _D), lambda i: (0, 0)),
        ],
        out_shape=[
            jax.ShapeDtypeStruct((_TOT, _D), jnp.float32),
            jax.ShapeDtypeStruct((8, _D), jnp.float32),
        ],
        scratch_shapes=[pltpu.VMEM((8, _D), jnp.float32)],
    )(g, ctr_rows, w0p, b0)


def _mlp2_body(y_ref, a_ref, d_ref, w_ref, b_ref, o_ref, st_ref, acc):
    i = pl.program_id(0)

    @pl.when(i == 0)
    def _():
        acc[...] = jnp.zeros_like(acc)

    z = jnp.maximum(y_ref[...] * a_ref[...] + d_ref[...], 0.0)
    yv = jnp.dot(z.astype(jnp.bfloat16), w_ref[...].astype(jnp.bfloat16),
                 preferred_element_type=jnp.float32) + b_ref[...]
    o_ref[...] = yv
    acc[0:1, :] = acc[0:1, :] + jnp.sum(yv, axis=0, keepdims=True)
    acc[1:2, :] = acc[1:2, :] + jnp.sum(yv * yv, axis=0, keepdims=True)

    @pl.when(i == _GRID1 - 1)
    def _():
        st_ref[...] = acc[...]


def _run_mlp2(y0, a0, d0, w1p, b1):
    return pl.pallas_call(
        _mlp2_body,
        grid=(_GRID1,),
        in_specs=[
            pl.BlockSpec((_ROWS, _D), lambda i: (i, 0)),
            pl.BlockSpec((1, _D), lambda i: (0, 0)),
            pl.BlockSpec((1, _D), lambda i: (0, 0)),
            pl.BlockSpec((_D, _D), lambda i: (0, 0)),
            pl.BlockSpec((1, _D), lambda i: (0, 0)),
        ],
        out_specs=[
            pl.BlockSpec((_ROWS, _D), lambda i: (i, 0)),
            pl.BlockSpec((8, _D), lambda i: (0, 0)),
        ],
        out_shape=[
            jax.ShapeDtypeStruct((_TOT, _D), jnp.float32),
            jax.ShapeDtypeStruct((8, _D), jnp.float32),
        ],
        scratch_shapes=[pltpu.VMEM((8, _D), jnp.float32)],
    )(y0, a0, d0, w1p, b1)


def _mlp3_body(y_ref, a_ref, d_ref, w_ref, b_ref, mx_ref, mn_ref, st_ref, acc):
    i = pl.program_id(0)

    @pl.when(i == 0)
    def _():
        acc[...] = jnp.zeros_like(acc)

    z = jnp.maximum(y_ref[...] * a_ref[...] + d_ref[...], 0.0)
    yv = jnp.dot(z.astype(jnp.bfloat16), w_ref[...].astype(jnp.bfloat16),
                 preferred_element_type=jnp.float32) + b_ref[...]
    acc[0:1, :] = acc[0:1, :] + jnp.sum(yv, axis=0, keepdims=True)
    acc[1:2, :] = acc[1:2, :] + jnp.sum(yv * yv, axis=0, keepdims=True)
    yr = yv.reshape(_ROWS // _S, _S, 64)
    mx_ref[...] = jnp.max(yr, axis=1)
    mn_ref[...] = jnp.min(yr, axis=1)

    @pl.when(i == _GRID1 - 1)
    def _():
        st_ref[...] = acc[...]


def _run_mlp3(y1, a1, d1, w2p, b2):
    return pl.pallas_call(
        _mlp3_body,
        grid=(_GRID1,),
        in_specs=[
            pl.BlockSpec((_ROWS, _D), lambda i: (i, 0)),
            pl.BlockSpec((1, _D), lambda i: (0, 0)),
            pl.BlockSpec((1, _D), lambda i: (0, 0)),
            pl.BlockSpec((_D, 64), lambda i: (0, 0)),
            pl.BlockSpec((1, 64), lambda i: (0, 0)),
        ],
        out_specs=[
            pl.BlockSpec((_ROWS // _S, 64), lambda i: (i, 0)),
            pl.BlockSpec((_ROWS // _S, 64), lambda i: (i, 0)),
            pl.BlockSpec((8, 64), lambda i: (0, 0)),
        ],
        out_shape=[
            jax.ShapeDtypeStruct((_B * _P, 64), jnp.float32),
            jax.ShapeDtypeStruct((_B * _P, 64), jnp.float32),
            jax.ShapeDtypeStruct((8, 64), jnp.float32),
        ],
        scratch_shapes=[pltpu.VMEM((8, 64), jnp.float32)],
    )(y1, a1, d1, w2p, b2)


def _fin_body(mx_ref, mn_ref, g_ref, a_ref, d_ref, o_ref):
    g = g_ref[...]
    sel = jnp.where(g >= 0.0, mx_ref[...], mn_ref[...])
    o_ref[...] = jnp.maximum(sel * a_ref[...] + d_ref[...], 0.0)


def _run_final(mx, mn, g2, a2, d2):
    return pl.pallas_call(
        _fin_body,
        grid=(16,),
        in_specs=[
            pl.BlockSpec((_ROWS, 64), lambda i: (i, 0)),
            pl.BlockSpec((_ROWS, 64), lambda i: (i, 0)),
            pl.BlockSpec((1, 64), lambda i: (0, 0)),
            pl.BlockSpec((1, 64), lambda i: (0, 0)),
            pl.BlockSpec((1, 64), lambda i: (0, 0)),
        ],
        out_specs=pl.BlockSpec((_ROWS, 64), lambda i: (i, 0)),
        out_shape=jax.ShapeDtypeStruct((_B * _P, 64), jnp.float32),
    )(mx, mn, g2, a2, d2)


def _bn_affine(s, ss, gamma, beta):
    cnt = float(_TOT)
    mean = s / cnt
    var = jnp.maximum(ss / cnt - mean * mean, 0.0)
    a = gamma / jnp.sqrt(var + _EPS)
    d = beta - mean * a
    return a, d


def kernel(xyz, points, w0, b0, gamma0, beta0, w1, b1, gamma1, beta1,
           w2, b2, gamma2, beta2):
    new_xyz = _run_fps(xyz)
    ctr_t = jnp.transpose(new_xyz, (0, 2, 1))

    idx = _run_ball_query(xyz, ctr_t)

    table = jnp.concatenate(
        [jnp.transpose(xyz, (0, 2, 1)),
         jnp.transpose(points, (0, 2, 1)),
         jnp.zeros((_B, _N, 128 - 19), jnp.float32)], axis=-1
    ).reshape(_B * _N, 128)
    g = _sc_gather(table, idx.reshape(_TOT))

    w0p = jnp.zeros((_D, _D), jnp.float32).at[:19, :].set(w0.T)
    w1p = w1.T
    w2p = w2.T
    ctr_rows = ctr_t.reshape(_B * _P, 3)
    y0, st0 = _run_mlp1(g, ctr_rows, w0p, b0.reshape(1, _D))
    a0, d0 = _bn_affine(st0[0], st0[1], gamma0, beta0)
    y1, st1 = _run_mlp2(y0, a0.reshape(1, _D), d0.reshape(1, _D),
                        w1p, b1.reshape(1, _D))
    a1, d1 = _bn_affine(st1[0], st1[1], gamma1, beta1)
    mx, mn, st2 = _run_mlp3(y1, a1.reshape(1, _D), d1.reshape(1, _D),
                            w2p, b2.reshape(1, 64))
    a2, d2 = _bn_affine(st2[0, :], st2[1, :], gamma2, beta2)
    pts = _run_final(mx, mn, gamma2.reshape(1, 64), a2.reshape(1, 64),
                     d2.reshape(1, 64))
    new_points = jnp.transpose(pts.reshape(_B, _P, 64), (0, 2, 1))
    return new_xyz, new_points


# speedup vs baseline: 12.5017x; 1.0076x over previous
import functools

import jax
import jax.numpy as jnp
from jax import lax
from jax.experimental import pallas as pl
from jax.experimental.pallas import tpu as pltpu
from jax.experimental.pallas import tpu_sc as plsc

_B, _N, _P, _S = 8, 8192, 1024, 32
_R2 = 0.4 * 0.4
_EPS = 1e-5
_NSUB, _NLANE = 8, 1024
_TOT = _B * _P * _S
_D = 32
_ROWS = 512
_GRID1 = _TOT // _ROWS
_NW = 32
_CHUNK = 128


def _fps_body(xyz_ref, ctr_ref):
    x = xyz_ref[0]
    y = xyz_ref[1]
    z = xyz_ref[2]
    iota_n = lax.broadcasted_iota(jnp.int32, (_B, _N), 1)
    iota_p = lax.broadcasted_iota(jnp.int32, (_B, _P), 1)

    def body(i, st):
        dists, far, ax, ay, az = st
        oh = iota_n == far
        cx = jnp.sum(jnp.where(oh, x, 0.0), axis=1, keepdims=True)
        cy = jnp.sum(jnp.where(oh, y, 0.0), axis=1, keepdims=True)
        cz = jnp.sum(jnp.where(oh, z, 0.0), axis=1, keepdims=True)
        rec = iota_p == i
        ax = jnp.where(rec, cx, ax)
        ay = jnp.where(rec, cy, ay)
        az = jnp.where(rec, cz, az)
        dx = x - cx
        dy = y - cy
        dz = z - cz
        d = dx * dx + dy * dy + dz * dz
        dists = jnp.minimum(dists, d)
        m = jnp.max(dists, axis=1, keepdims=True)
        far = jnp.min(jnp.where(dists == m, iota_n, _N), axis=1,
                      keepdims=True).astype(jnp.int32)
        return dists, far, ax, ay, az

    init = (jnp.full((_B, _N), 1e10, jnp.float32),
            jnp.zeros((_B, 1), jnp.int32),
            jnp.zeros((_B, _P), jnp.float32),
            jnp.zeros((_B, _P), jnp.float32),
            jnp.zeros((_B, _P), jnp.float32))
    _, _, ax, ay, az = lax.fori_loop(0, _P, body, init)
    ctr_ref[0] = ax
    ctr_ref[1] = ay
    ctr_ref[2] = az


def _run_fps(xyz):
    xyz_sb = jnp.transpose(xyz, (1, 0, 2))
    ctr = pl.pallas_call(
        _fps_body,
        grid=(1,),
        in_specs=[pl.BlockSpec((3, _B, _N), lambda i: (0, 0, 0))],
        out_specs=pl.BlockSpec((3, _B, _P), lambda i: (0, 0, 0)),
        out_shape=jax.ShapeDtypeStruct((3, _B, _P), jnp.float32),
    )(xyz_sb)
    return jnp.transpose(ctr, (1, 0, 2))


_PT = 64


def _bq_body(xyz_ref, ctr_ref, idx_ref):
    b = pl.program_id(0)
    x = xyz_ref[0, 0:1, :]
    y = xyz_ref[0, 1:2, :]
    z = xyz_ref[0, 2:3, :]
    cx = ctr_ref[0, :, 0:1]
    cy = ctr_ref[0, :, 1:2]
    cz = ctr_ref[0, :, 2:3]
    xx = x * x + y * y + z * z
    cc = cx * cx + cy * cy + cz * cz
    xb = x.astype(jnp.bfloat16).astype(jnp.float32)
    yb = y.astype(jnp.bfloat16).astype(jnp.float32)
    zb = z.astype(jnp.bfloat16).astype(jnp.float32)
    cxb = cx.astype(jnp.bfloat16).astype(jnp.float32)
    cyb = cy.astype(jnp.bfloat16).astype(jnp.float32)
    czb = cz.astype(jnp.bfloat16).astype(jnp.float32)
    cross = cxb * xb + cyb * yb + czb * zb
    d2 = (cc + xx) - 2.0 * cross
    iota = lax.broadcasted_iota(jnp.int32, (_PT, _N), 1)
    cand = jnp.where(d2 <= _R2, iota, _N)

    def _rowmin(v):
        parts = [v[:, k * 512:(k + 1) * 512] for k in range(v.shape[1] // 512)]
        while len(parts) > 1:
            parts = [jnp.minimum(parts[i], parts[i + 1])
                     for i in range(0, len(parts), 2)]
        return jnp.min(parts[0], axis=1, keepdims=True)

    m = _rowmin(cand)
    cols = [m]
    for _ in range(_S - 1):
        m = _rowmin(jnp.where(cand > m, cand, _N))
        cols.append(m)
    idxt = jnp.concatenate(cols, axis=1)
    idxt = jnp.where(idxt == _N, idxt[:, 0:1], idxt)
    idxt = jnp.minimum(idxt, _N - 1)
    idx_ref[0] = idxt + b * _N


def _run_ball_query(xyz, ctr_t):
    return pl.pallas_call(
        _bq_body,
        grid=(_B, _P // _PT),
        in_specs=[
            pl.BlockSpec((1, 3, _N), lambda b, t: (b, 0, 0)),
            pl.BlockSpec((1, _PT, 3), lambda b, t: (b, t, 0)),
        ],
        out_specs=pl.BlockSpec((1, _PT, _S), lambda b, t: (b, t, 0)),
        out_shape=jax.ShapeDtypeStruct((_B, _P, _S), jnp.int32),
    )(xyz, ctr_t)


def _sc_gather_body(tab_ref, idx_ref, out_ref, idx_v0, idx_v1,
                    rows_v0, rows_v1, sem0, sem1):
    wid = lax.axis_index("s") * 2 + lax.axis_index("c")
    per_w = _TOT // _NW
    base = wid * per_w

    def chunk2(c, carry):
        off0 = base + (2 * c) * _CHUNK
        off1 = off0 + _CHUNK
        pltpu.sync_copy(idx_ref.at[pl.ds(off0, _CHUNK)], idx_v0)
        g0 = pltpu.async_copy(tab_ref.at[idx_v0], rows_v0, sem0)
        pltpu.sync_copy(idx_ref.at[pl.ds(off1, _CHUNK)], idx_v1)
        g0.wait()
        g1 = pltpu.async_copy(tab_ref.at[idx_v1], rows_v1, sem1)
        pltpu.sync_copy(rows_v0, out_ref.at[pl.ds(off0, _CHUNK)])
        g1.wait()
        pltpu.sync_copy(rows_v1, out_ref.at[pl.ds(off1, _CHUNK)])
        return carry

    lax.fori_loop(0, per_w // (2 * _CHUNK), chunk2, 0)


def _sc_gather(table, flat_idx):
    mesh = plsc.VectorSubcoreMesh(core_axis_name="c", subcore_axis_name="s")
    f = functools.partial(
        pl.kernel,
        mesh=mesh,
        out_type=jax.ShapeDtypeStruct((_TOT, 128), jnp.float32),
        scratch_types=[
            pltpu.VMEM((_CHUNK,), jnp.int32),
            pltpu.VMEM((_CHUNK,), jnp.int32),
            pltpu.VMEM((_CHUNK, 128), jnp.float32),
            pltpu.VMEM((_CHUNK, 128), jnp.float32),
            pltpu.SemaphoreType.DMA,
            pltpu.SemaphoreType.DMA,
        ],
    )(_sc_gather_body)
    return f(table, flat_idx)


def _mlp1_body(g_ref, c_ref, w_ref, b_ref, y_ref, st_ref, acc):
    i = pl.program_id(0)

    @pl.when(i == 0)
    def _():
        acc[...] = jnp.zeros_like(acc)

    g = g_ref[...][:, 0:_D]
    c = c_ref[...]
    ce = jnp.reshape(jnp.broadcast_to(c[:, None, :], (_ROWS // _S, _S, 3)),
                     (_ROWS, 3))
    xcat = jnp.concatenate([g[:, 0:3] - ce, g[:, 3:]], axis=1)
    yv = jnp.dot(xcat.astype(jnp.bfloat16), w_ref[...].astype(jnp.bfloat16),
                 preferred_element_type=jnp.float32) + b_ref[...]
    y_ref[...] = yv
    acc[0:1, :] = acc[0:1, :] + jnp.sum(yv, axis=0, keepdims=True)
    acc[1:2, :] = acc[1:2, :] + jnp.sum(yv * yv, axis=0, keepdims=True)

    @pl.when(i == _GRID1 - 1)
    def _():
        st_ref[...] = acc[...]


def _run_mlp1(g, ctr_rows, w0p, b0):
    return pl.pallas_call(
        _mlp1_body,
        grid=(_GRID1,),
        in_specs=[
            pl.BlockSpec((_ROWS, 128), lambda i: (i, 0)),
            pl.BlockSpec((_ROWS // _S, 3), lambda i: (i, 0)),
            pl.BlockSpec((_D, _D), lambda i: (0, 0)),
            pl.BlockSpec((1, _D), lambda i: (0, 0)),
        ],
        out_specs=[
            pl.BlockSpec((_ROWS, _D), lambda i: (i, 0)),
            pl.BlockSpec((8, _D), lambda i: (0, 0)),
        ],
        out_shape=[
            jax.ShapeDtypeStruct((_TOT, _D), jnp.float32),
            jax.ShapeDtypeStruct((8, _D), jnp.float32),
        ],
        scratch_shapes=[pltpu.VMEM((8, _D), jnp.float32)],
    )(g, ctr_rows, w0p, b0)


def _mlp2_body(y_ref, a_ref, d_ref, w_ref, b_ref, o_ref, st_ref, acc):
    i = pl.program_id(0)

    @pl.when(i == 0)
    def _():
        acc[...] = jnp.zeros_like(acc)

    z = jnp.maximum(y_ref[...] * a_ref[...] + d_ref[...], 0.0)
    yv = jnp.dot(z.astype(jnp.bfloat16), w_ref[...].astype(jnp.bfloat16),
                 preferred_element_type=jnp.float32) + b_ref[...]
    o_ref[...] = yv
    acc[0:1, :] = acc[0:1, :] + jnp.sum(yv, axis=0, keepdims=True)
    acc[1:2, :] = acc[1:2, :] + jnp.sum(yv * yv, axis=0, keepdims=True)

    @pl.when(i == _GRID1 - 1)
    def _():
        st_ref[...] = acc[...]


def _run_mlp2(y0, a0, d0, w1p, b1):
    return pl.pallas_call(
        _mlp2_body,
        grid=(_GRID1,),
        in_specs=[
            pl.BlockSpec((_ROWS, _D), lambda i: (i, 0)),
            pl.BlockSpec((1, _D), lambda i: (0, 0)),
            pl.BlockSpec((1, _D), lambda i: (0, 0)),
            pl.BlockSpec((_D, _D), lambda i: (0, 0)),
            pl.BlockSpec((1, _D), lambda i: (0, 0)),
        ],
        out_specs=[
            pl.BlockSpec((_ROWS, _D), lambda i: (i, 0)),
            pl.BlockSpec((8, _D), lambda i: (0, 0)),
        ],
        out_shape=[
            jax.ShapeDtypeStruct((_TOT, _D), jnp.float32),
            jax.ShapeDtypeStruct((8, _D), jnp.float32),
        ],
        scratch_shapes=[pltpu.VMEM((8, _D), jnp.float32)],
    )(y0, a0, d0, w1p, b1)


def _mlp3_body(y_ref, a_ref, d_ref, w_ref, b_ref, mx_ref, mn_ref, st_ref, acc):
    i = pl.program_id(0)

    @pl.when(i == 0)
    def _():
        acc[...] = jnp.zeros_like(acc)

    z = jnp.maximum(y_ref[...] * a_ref[...] + d_ref[...], 0.0)
    yv = jnp.dot(z.astype(jnp.bfloat16), w_ref[...].astype(jnp.bfloat16),
                 preferred_element_type=jnp.float32) + b_ref[...]
    acc[0:1, :] = acc[0:1, :] + jnp.sum(yv, axis=0, keepdims=True)
    acc[1:2, :] = acc[1:2, :] + jnp.sum(yv * yv, axis=0, keepdims=True)
    yr = yv.reshape(_ROWS // _S, _S, 64)
    mx_ref[...] = jnp.max(yr, axis=1)
    mn_ref[...] = jnp.min(yr, axis=1)

    @pl.when(i == _GRID1 - 1)
    def _():
        st_ref[...] = acc[...]


def _run_mlp3(y1, a1, d1, w2p, b2):
    return pl.pallas_call(
        _mlp3_body,
        grid=(_GRID1,),
        in_specs=[
            pl.BlockSpec((_ROWS, _D), lambda i: (i, 0)),
            pl.BlockSpec((1, _D), lambda i: (0, 0)),
            pl.BlockSpec((1, _D), lambda i: (0, 0)),
            pl.BlockSpec((_D, 64), lambda i: (0, 0)),
            pl.BlockSpec((1, 64), lambda i: (0, 0)),
        ],
        out_specs=[
            pl.BlockSpec((_ROWS // _S, 64), lambda i: (i, 0)),
            pl.BlockSpec((_ROWS // _S, 64), lambda i: (i, 0)),
            pl.BlockSpec((8, 64), lambda i: (0, 0)),
        ],
        out_shape=[
            jax.ShapeDtypeStruct((_B * _P, 64), jnp.float32),
            jax.ShapeDtypeStruct((_B * _P, 64), jnp.float32),
            jax.ShapeDtypeStruct((8, 64), jnp.float32),
        ],
        scratch_shapes=[pltpu.VMEM((8, 64), jnp.float32)],
    )(y1, a1, d1, w2p, b2)


def _fin_body(mx_ref, mn_ref, g_ref, a_ref, d_ref, o_ref):
    g = g_ref[...]
    sel = jnp.where(g >= 0.0, mx_ref[...], mn_ref[...])
    o_ref[...] = jnp.maximum(sel * a_ref[...] + d_ref[...], 0.0)


def _run_final(mx, mn, g2, a2, d2):
    return pl.pallas_call(
        _fin_body,
        grid=(16,),
        in_specs=[
            pl.BlockSpec((_ROWS, 64), lambda i: (i, 0)),
            pl.BlockSpec((_ROWS, 64), lambda i: (i, 0)),
            pl.BlockSpec((1, 64), lambda i: (0, 0)),
            pl.BlockSpec((1, 64), lambda i: (0, 0)),
            pl.BlockSpec((1, 64), lambda i: (0, 0)),
        ],
        out_specs=pl.BlockSpec((_ROWS, 64), lambda i: (i, 0)),
        out_shape=jax.ShapeDtypeStruct((_B * _P, 64), jnp.float32),
    )(mx, mn, g2, a2, d2)


def _bn_affine(s, ss, gamma, beta):
    cnt = float(_TOT)
    mean = s / cnt
    var = jnp.maximum(ss / cnt - mean * mean, 0.0)
    a = gamma / jnp.sqrt(var + _EPS)
    d = beta - mean * a
    return a, d


def kernel(xyz, points, w0, b0, gamma0, beta0, w1, b1, gamma1, beta1,
           w2, b2, gamma2, beta2):
    new_xyz = _run_fps(xyz)
    ctr_t = jnp.transpose(new_xyz, (0, 2, 1))

    idx = _run_ball_query(xyz, ctr_t)

    table = jnp.concatenate(
        [jnp.transpose(xyz, (0, 2, 1)),
         jnp.transpose(points, (0, 2, 1)),
         jnp.zeros((_B, _N, 128 - 19), jnp.float32)], axis=-1
    ).reshape(_B * _N, 128)
    g = _sc_gather(table, idx.reshape(_TOT))

    w0p = jnp.zeros((_D, _D), jnp.float32).at[:19, :].set(w0.T)
    w1p = w1.T
    w2p = w2.T
    ctr_rows = ctr_t.reshape(_B * _P, 3)
    y0, st0 = _run_mlp1(g, ctr_rows, w0p, b0.reshape(1, _D))
    a0, d0 = _bn_affine(st0[0], st0[1], gamma0, beta0)
    y1, st1 = _run_mlp2(y0, a0.reshape(1, _D), d0.reshape(1, _D),
                        w1p, b1.reshape(1, _D))
    a1, d1 = _bn_affine(st1[0], st1[1], gamma1, beta1)
    mx, mn, st2 = _run_mlp3(y1, a1.reshape(1, _D), d1.reshape(1, _D),
                            w2p, b2.reshape(1, 64))
    a2, d2 = _bn_affine(st2[0, :], st2[1, :], gamma2, beta2)
    pts = _run_final(mx, mn, gamma2.reshape(1, 64), a2.reshape(1, 64),
                     d2.reshape(1, 64))
    new_points = jnp.transpose(pts.reshape(_B, _P, 64), (0, 2, 1))
    return new_xyz, new_points
